# baseline (speedup 1.0000x reference)
import functools

import jax
import jax.numpy as jnp
from jax import lax
from jax.experimental import pallas as pl
from jax.experimental.pallas import tpu as pltpu
from jax.experimental.pallas import tpu_sc as plsc


def _sc_gather_t(pos_t_flat, table_t_flat, seq, bsz, v, de):
    nw = 32
    per_b = bsz // nw
    mesh = plsc.VectorSubcoreMesh(core_axis_name="c", subcore_axis_name="s")

    @functools.partial(
        pl.kernel,
        mesh=mesh,
        compiler_params=pltpu.CompilerParams(needs_layout_passes=False),
        out_type=jax.ShapeDtypeStruct((seq * de // 2, bsz), jnp.int32),
        scratch_types=[
            pltpu.VMEM((v * de,), jnp.float32),
            pltpu.VMEM((per_b,), jnp.int32),
            pltpu.VMEM((per_b,), jnp.int32),
            pltpu.VMEM((de // 2, per_b), jnp.int32),
            pltpu.VMEM((de // 2, per_b), jnp.int32),
            pltpu.SemaphoreType.DMA,
            pltpu.SemaphoreType.DMA,
            pltpu.SemaphoreType.DMA,
            pltpu.SemaphoreType.DMA,
        ],
    )
    def gather_kernel(pos_hbm, tab_hbm, out_hbm, tab_v, idx0, idx1,
                      rows0, rows1, si0, si1, so0, so1):
        wid = lax.axis_index("s") * 2 + lax.axis_index("c")
        b0 = wid * per_b
        pltpu.sync_copy(tab_hbm, tab_v)
        pltpu.async_copy(pos_hbm.at[pl.ds(b0, per_b)], idx0, si0)

        def pair_body(i, carry):
            for p in (0, 1):
                s = 2 * i + p
                idx_v = (idx0, idx1)[p]
                rows_v = (rows0, rows1)[p]
                si = (si0, si1)[p]
                so = (so0, so1)[p]
                idx_n = (idx1, idx0)[p]
                si_n = (si1, si0)[p]

                @pl.when(s + 1 < seq)
                def _():
                    pltpu.async_copy(
                        pos_hbm.at[pl.ds((s + 1) * bsz + b0, per_b)], idx_n, si_n
                    )

                pltpu.make_async_copy(
                    pos_hbm.at[pl.ds(s * bsz + b0, per_b)], idx_v, si
                ).wait()

                @pl.when(s >= 2)
                def _():
                    pltpu.make_async_copy(
                        rows_v, out_hbm.at[pl.ds(0, de // 2), pl.ds(b0, per_b)], so
                    ).wait()

                @plsc.parallel_loop(0, per_b // 16, unroll=2)
                def grp(g):
                    idx16 = idx_v[pl.ds(g * 16, 16)]

                    def rne(val):
                        t = plsc.bitcast(val, jnp.int32)
                        rb = lax.shift_right_logical(t, 16) & 1
                        return lax.shift_right_logical(t + 0x7FFF + rb, 16)

                    for e2 in range(de // 2):
                        v0 = plsc.load_gather(tab_v, [idx16 + (2 * e2) * v])
                        v1 = plsc.load_gather(tab_v, [idx16 + (2 * e2 + 1) * v])
                        rows_v[e2, pl.ds(g * 16, 16)] = (
                            lax.shift_left(rne(v1), 16) | rne(v0)
                        )

                pltpu.async_copy(
                    rows_v,
                    out_hbm.at[pl.ds(s * (de // 2), de // 2), pl.ds(b0, per_b)],
                    so,
                )
            return carry

        lax.fori_loop(0, seq // 2, pair_body, 0)
        pltpu.make_async_copy(
            rows0, out_hbm.at[pl.ds(0, de // 2), pl.ds(b0, per_b)], so0
        ).wait()
        pltpu.make_async_copy(
            rows1, out_hbm.at[pl.ds(0, de // 2), pl.ds(b0, per_b)], so1
        ).wait()

    return gather_kernel(pos_t_flat, table_t_flat)


def _tc_combine_t(x_t2, pe_packed, w2, b128, seq, bsz, di, de):
    bl = 4096
    h = de // 2

    def body(x_ref, pe_ref, w_ref, b_ref, o_ref):
        xb = x_ref[...]
        acc_lo = jnp.dot(w_ref[0:h], xb, preferred_element_type=jnp.float32)
        acc_hi = jnp.dot(w_ref[h:de], xb, preferred_element_type=jnp.float32)
        pew = pe_ref[...]
        pe_lo = lax.bitcast_convert_type(lax.shift_left(pew, 16), jnp.float32)
        pe_hi = lax.bitcast_convert_type(pew & jnp.int32(-65536), jnp.float32)
        out_lo = acc_lo + b_ref[0:h, 0:1] + pe_lo
        out_hi = acc_hi + b_ref[h:de, 0:1] + pe_hi
        o_ref[...] = jnp.stack([out_lo, out_hi], axis=1).reshape(de, bl)

    return pl.pallas_call(
        body,
        grid=(seq, bsz // bl),
        in_specs=[
            pl.BlockSpec((di, bl), lambda s, l: (s, l)),
            pl.BlockSpec((h, bl), lambda s, l: (s, l)),
            pl.BlockSpec((de, di), lambda s, l: (0, 0)),
            pl.BlockSpec((de, 128), lambda s, l: (0, 0)),
        ],
        out_specs=pl.BlockSpec((de, bl), lambda s, l: (s, l)),
        out_shape=jax.ShapeDtypeStruct((seq * de, bsz), jnp.float32),
    )(x_t2, pe_packed, w2, b128)


def kernel(x, positions, pos_table, W, b):
    bsz, seq, di = x.shape
    v, de = pos_table.shape

    x_t2 = x.transpose(1, 2, 0).reshape(seq * di, bsz)
    pos_t_flat = positions.T.astype(jnp.int32).reshape(seq * bsz)
    table_t_flat = pos_table.T.reshape(de * v)
    w2 = jnp.concatenate([W[0::2], W[1::2]], axis=0)
    b2 = jnp.concatenate([b[0::2], b[1::2]])
    b128 = jnp.broadcast_to(b2.reshape(de, 1), (de, 128))

    pe_packed = _sc_gather_t(pos_t_flat, table_t_flat, seq, bsz, v, de)
    out_t2 = _tc_combine_t(x_t2, pe_packed, w2, b128, seq, bsz, di, de)
    return out_t2.reshape(seq, de, bsz).transpose(2, 0, 1)


# speedup vs baseline: 1.4227x; 1.4227x over previous
import functools

import jax
import jax.numpy as jnp
from jax import lax
from jax.experimental import pallas as pl
from jax.experimental.pallas import tpu as pltpu
from jax.experimental.pallas import tpu_sc as plsc


def _sc_gather_t(pos_t2d, table_t_flat, seq, bsz, v, de):
    nw = 32
    per_b = bsz // nw
    np_ = seq // 2
    mesh = plsc.VectorSubcoreMesh(core_axis_name="c", subcore_axis_name="s")

    @functools.partial(
        pl.kernel,
        mesh=mesh,
        compiler_params=pltpu.CompilerParams(needs_layout_passes=False),
        out_type=jax.ShapeDtypeStruct((np_ * de, bsz), jnp.int32),
        scratch_types=[
            pltpu.VMEM((v * de,), jnp.float32),
            pltpu.VMEM((2, per_b), jnp.int32),
            pltpu.VMEM((2, per_b), jnp.int32),
            pltpu.VMEM((de, per_b), jnp.int32),
            pltpu.VMEM((de, per_b), jnp.int32),
            pltpu.SemaphoreType.DMA,
            pltpu.SemaphoreType.DMA,
            pltpu.SemaphoreType.DMA,
            pltpu.SemaphoreType.DMA,
        ],
    )
    def gather_kernel(pos_hbm, tab_hbm, out_hbm, tab_v, idx0, idx1,
                      rows0, rows1, si0, si1, so0, so1):
        wid = lax.axis_index("s") * 2 + lax.axis_index("c")
        b0 = wid * per_b
        pltpu.sync_copy(tab_hbm, tab_v)
        pltpu.async_copy(pos_hbm.at[pl.ds(0, 2), pl.ds(b0, per_b)], idx0, si0)

        def pair_body(i, carry):
            for p in (0, 1):
                pp = 2 * i + p
                idx_v = (idx0, idx1)[p]
                rows_v = (rows0, rows1)[p]
                si = (si0, si1)[p]
                so = (so0, so1)[p]
                idx_n = (idx1, idx0)[p]
                si_n = (si1, si0)[p]

                @pl.when(pp + 1 < np_)
                def _():
                    pltpu.async_copy(
                        pos_hbm.at[pl.ds((pp + 1) * 2, 2), pl.ds(b0, per_b)],
                        idx_n,
                        si_n,
                    )

                pltpu.make_async_copy(
                    pos_hbm.at[pl.ds(pp * 2, 2), pl.ds(b0, per_b)], idx_v, si
                ).wait()

                @pl.when(pp >= 2)
                def _():
                    pltpu.make_async_copy(
                        rows_v, out_hbm.at[pl.ds(0, de), pl.ds(b0, per_b)], so
                    ).wait()

                @plsc.parallel_loop(0, per_b // 16, unroll=2)
                def grp(g):
                    ia = idx_v[0, pl.ds(g * 16, 16)]
                    ib = idx_v[1, pl.ds(g * 16, 16)]
                    for e in range(de):
                        v0 = plsc.load_gather(tab_v, [ia + e * v])
                        v1 = plsc.load_gather(tab_v, [ib + e * v])
                        t0 = plsc.bitcast(v0, jnp.int32) + 0x8000
                        t1 = plsc.bitcast(v1, jnp.int32) + 0x8000
                        rows_v[e, pl.ds(g * 16, 16)] = (
                            t1 & jnp.int32(-65536)
                        ) | lax.shift_right_logical(t0, 16)

                pltpu.async_copy(
                    rows_v, out_hbm.at[pl.ds(pp * de, de), pl.ds(b0, per_b)], so
                )
            return carry

        lax.fori_loop(0, np_ // 2, pair_body, 0)
        pltpu.make_async_copy(
            rows0, out_hbm.at[pl.ds(0, de), pl.ds(b0, per_b)], so0
        ).wait()
        pltpu.make_async_copy(
            rows1, out_hbm.at[pl.ds(0, de), pl.ds(b0, per_b)], so1
        ).wait()

    return gather_kernel(pos_t2d, table_t_flat)


def _tc_combine_t(x_t2, pe_packed, wbig, b128, seq, bsz, di, de):
    bl = 4096

    def body(x_ref, pe_ref, w_ref, b_ref, o_ref):
        acc = jnp.dot(w_ref[...], x_ref[...], preferred_element_type=jnp.float32)
        pew = pe_ref[...]
        pe_lo = lax.bitcast_convert_type(lax.shift_left(pew, 16), jnp.float32)
        pe_hi = lax.bitcast_convert_type(pew & jnp.int32(-65536), jnp.float32)
        o_ref[...] = (
            acc + b_ref[:, 0:1] + jnp.concatenate([pe_lo, pe_hi], axis=0)
        )

    return pl.pallas_call(
        body,
        grid=(seq // 2, bsz // bl),
        in_specs=[
            pl.BlockSpec((2 * di, bl), lambda s, l: (s, l)),
            pl.BlockSpec((de, bl), lambda s, l: (s, l)),
            pl.BlockSpec((2 * de, 2 * di), lambda s, l: (0, 0)),
            pl.BlockSpec((2 * de, 128), lambda s, l: (0, 0)),
        ],
        out_specs=pl.BlockSpec((2 * de, bl), lambda s, l: (s, l)),
        out_shape=jax.ShapeDtypeStruct((seq * de, bsz), jnp.float32),
    )(x_t2, pe_packed, wbig, b128)


def kernel(x, positions, pos_table, W, b):
    bsz, seq, di = x.shape
    v, de = pos_table.shape

    x_t2 = x.transpose(1, 2, 0).reshape(seq * di, bsz)
    pos_t2d = positions.T.astype(jnp.int32)
    table_t_flat = pos_table.T.reshape(de * v)
    z = jnp.zeros_like(W)
    wbig = jnp.concatenate(
        [jnp.concatenate([W, z], axis=1), jnp.concatenate([z, W], axis=1)],
        axis=0,
    )
    b128 = jnp.broadcast_to(
        jnp.concatenate([b, b]).reshape(2 * de, 1), (2 * de, 128)
    )

    pe_packed = _sc_gather_t(pos_t2d, table_t_flat, seq, bsz, v, de)
    out_t2 = _tc_combine_t(x_t2, pe_packed, wbig, b128, seq, bsz, di, de)
    return out_t2.reshape(seq, de, bsz).transpose(2, 0, 1)


# speedup vs baseline: 1.6207x; 1.1392x over previous
import functools

import jax
import jax.numpy as jnp
from jax import lax
from jax.experimental import pallas as pl
from jax.experimental.pallas import tpu as pltpu
from jax.experimental.pallas import tpu_sc as plsc


def _sc_gather_t(pos_t2d, table_t_flat, seq, bsz, v, de):
    nw = 32
    per_b = bsz // nw
    np_ = seq // 2
    mesh = plsc.VectorSubcoreMesh(core_axis_name="c", subcore_axis_name="s")

    @functools.partial(
        pl.kernel,
        mesh=mesh,
        compiler_params=pltpu.CompilerParams(needs_layout_passes=False),
        out_type=jax.ShapeDtypeStruct((np_ * de, bsz), jnp.int32),
        scratch_types=[
            pltpu.VMEM((v * de,), jnp.float32),
            pltpu.VMEM((2, per_b), jnp.int32),
            pltpu.VMEM((2, per_b), jnp.int32),
            pltpu.VMEM((de, per_b), jnp.int32),
            pltpu.VMEM((de, per_b), jnp.int32),
            pltpu.SemaphoreType.DMA,
            pltpu.SemaphoreType.DMA,
            pltpu.SemaphoreType.DMA,
            pltpu.SemaphoreType.DMA,
        ],
    )
    def gather_kernel(pos_hbm, tab_hbm, out_hbm, tab_v, idx0, idx1,
                      rows0, rows1, si0, si1, so0, so1):
        wid = lax.axis_index("s") * 2 + lax.axis_index("c")
        b0 = wid * per_b
        pltpu.sync_copy(tab_hbm, tab_v)
        pltpu.async_copy(pos_hbm.at[pl.ds(0, 2), pl.ds(b0, per_b)], idx0, si0)

        def pair_body(i, carry):
            for p in (0, 1):
                pp = 2 * i + p
                idx_v = (idx0, idx1)[p]
                rows_v = (rows0, rows1)[p]
                si = (si0, si1)[p]
                so = (so0, so1)[p]
                idx_n = (idx1, idx0)[p]
                si_n = (si1, si0)[p]

                @pl.when(pp + 1 < np_)
                def _():
                    pltpu.async_copy(
                        pos_hbm.at[pl.ds((pp + 1) * 2, 2), pl.ds(b0, per_b)],
                        idx_n,
                        si_n,
                    )

                pltpu.make_async_copy(
                    pos_hbm.at[pl.ds(pp * 2, 2), pl.ds(b0, per_b)], idx_v, si
                ).wait()

                @pl.when(pp >= 2)
                def _():
                    pltpu.make_async_copy(
                        rows_v, out_hbm.at[pl.ds(0, de), pl.ds(b0, per_b)], so
                    ).wait()

                @plsc.parallel_loop(0, per_b // 16, unroll=4)
                def grp(g):
                    ia = idx_v[0, pl.ds(g * 16, 16)]
                    ib = idx_v[1, pl.ds(g * 16, 16)]
                    for e in range(de):
                        v0 = plsc.load_gather(tab_v, [ia + e * v])
                        v1 = plsc.load_gather(tab_v, [ib + e * v])
                        t0 = plsc.bitcast(v0, jnp.int32) + 0x8000
                        t1 = plsc.bitcast(v1, jnp.int32) + 0x8000
                        rows_v[e, pl.ds(g * 16, 16)] = (
                            t1 & jnp.int32(-65536)
                        ) | lax.shift_right_logical(t0, 16)

                pltpu.async_copy(
                    rows_v, out_hbm.at[pl.ds(pp * de, de), pl.ds(b0, per_b)], so
                )
            return carry

        lax.fori_loop(0, np_ // 2, pair_body, 0)
        pltpu.make_async_copy(
            rows0, out_hbm.at[pl.ds(0, de), pl.ds(b0, per_b)], so0
        ).wait()
        pltpu.make_async_copy(
            rows1, out_hbm.at[pl.ds(0, de), pl.ds(b0, per_b)], so1
        ).wait()

    return gather_kernel(pos_t2d, table_t_flat)


def _tc_combine_t(x_t2, pe_packed, wbig, b128, seq, bsz, di, de):
    bl = 4096

    def body(x_ref, pe_ref, w_ref, b_ref, o_ref):
        acc = jnp.dot(w_ref[...], x_ref[...], preferred_element_type=jnp.float32)
        pew = pe_ref[...]
        pe_lo = lax.bitcast_convert_type(lax.shift_left(pew, 16), jnp.float32)
        pe_hi = lax.bitcast_convert_type(pew & jnp.int32(-65536), jnp.float32)
        o_ref[...] = (
            acc + b_ref[:, 0:1] + jnp.concatenate([pe_lo, pe_hi], axis=0)
        )

    return pl.pallas_call(
        body,
        grid=(seq // 2, bsz // bl),
        in_specs=[
            pl.BlockSpec((2 * di, bl), lambda s, l: (s, l)),
            pl.BlockSpec((de, bl), lambda s, l: (s, l)),
            pl.BlockSpec((2 * de, 2 * di), lambda s, l: (0, 0)),
            pl.BlockSpec((2 * de, 128), lambda s, l: (0, 0)),
        ],
        out_specs=pl.BlockSpec((2 * de, bl), lambda s, l: (s, l)),
        out_shape=jax.ShapeDtypeStruct((seq * de, bsz), jnp.float32),
    )(x_t2, pe_packed, wbig, b128)


def kernel(x, positions, pos_table, W, b):
    bsz, seq, di = x.shape
    v, de = pos_table.shape

    x_t2 = x.transpose(1, 2, 0).reshape(seq * di, bsz)
    pos_t2d = positions.T.astype(jnp.int32)
    table_t_flat = pos_table.T.reshape(de * v)
    z = jnp.zeros_like(W)
    wbig = jnp.concatenate(
        [jnp.concatenate([W, z], axis=1), jnp.concatenate([z, W], axis=1)],
        axis=0,
    )
    b128 = jnp.broadcast_to(
        jnp.concatenate([b, b]).reshape(2 * de, 1), (2 * de, 128)
    )

    pe_packed = _sc_gather_t(pos_t2d, table_t_flat, seq, bsz, v, de)
    out_t2 = _tc_combine_t(x_t2, pe_packed, wbig, b128, seq, bsz, di, de)
    return out_t2.reshape(seq, de, bsz).transpose(2, 0, 1)
